# baseline (speedup 1.0000x reference)
import jax
import jax.numpy as jnp
from jax import lax
from jax.experimental import pallas as pl
from jax.experimental.pallas import tpu as pltpu
from jax.experimental.pallas import tpu_sc as plsc

NC = 2
NS = 16
L = 16
NW = NC * NS

D = 192
MW = 128
TW = D - MW
DP = 256
B = 16384
BPW = B // NW
CH = 64
NP = BPW // (2 * CH)


def _body(gu_hbm, gi_hbm, user_hbm, item_hbm,
          xui_hbm, gu_out, gi_out,
          idx_u, idx_i, ru0, ru1, ri0, ri1, tu0, tu1, ti0, ti1,
          xv, accb, padu, padi, s0, s1):
    cid = lax.axis_index("c")
    sid = lax.axis_index("s")
    wid = sid * NC + cid
    base = wid * BPW

    pltpu.sync_copy(user_hbm.at[pl.ds(base, BPW)], idx_u)
    pltpu.sync_copy(item_hbm.at[pl.ds(base, BPW)], idx_i)

    ru = (ru0, ru1)
    ri = (ri0, ri1)
    tu = (tu0, tu1)
    ti = (ti0, ti1)
    sems = (s0, s1)

    lane = lax.iota(jnp.int32, L)
    lane17 = lane * 17

    z = jnp.zeros((L,), jnp.float32)

    def zrow(r, carry):
        for j in range(TW // L, 128 // L):
            padu[r, pl.ds(j * L, L)] = z
            padi[r, pl.ds(j * L, L)] = z
        return carry
    lax.fori_loop(0, CH, zrow, 0)

    def issue(off, b):
        h = [
            pltpu.async_copy(
                gu_hbm.at[idx_u.at[pl.ds(off, CH)], pl.ds(0, MW)],
                ru[b], sems[b]),
            pltpu.async_copy(
                gi_hbm.at[idx_i.at[pl.ds(off, CH)], pl.ds(0, MW)],
                ri[b], sems[b]),
        ]
        for gg in range(CH // L):
            rvu = idx_u[pl.ds(off + gg * L, L)]
            rvi = idx_i[pl.ds(off + gg * L, L)]
            for l in range(L):
                kk = gg * L + l
                h.append(pltpu.async_copy(
                    gu_hbm.at[pl.ds(rvu[l], 1), pl.ds(MW, TW)],
                    tu[b].at[pl.ds(kk, 1)], sems[b]))
                h.append(pltpu.async_copy(
                    gi_hbm.at[pl.ds(rvi[l], 1), pl.ds(MW, TW)],
                    ti[b].at[pl.ds(kk, 1)], sems[b]))
        return h

    def process(off, b):
        def prow(r, carry):
            for j in range(TW // L):
                padu[r, pl.ds(j * L, L)] = tu[b][r, pl.ds(j * L, L)]
                padi[r, pl.ds(j * L, L)] = ti[b][r, pl.ds(j * L, L)]
            return carry
        lax.fori_loop(0, CH, prow, 0)
        pltpu.sync_copy(ru[b], gu_out.at[pl.ds(base + off, CH), pl.ds(0, MW)])
        pltpu.sync_copy(padu, gu_out.at[pl.ds(base + off, CH), pl.ds(MW, 128)])
        pltpu.sync_copy(ri[b], gi_out.at[pl.ds(base + off, CH), pl.ds(0, MW)])
        pltpu.sync_copy(padi, gi_out.at[pl.ds(base + off, CH), pl.ds(MW, 128)])

        def group(g, carry):
            for l in range(L):
                r = g * L + l
                acc = ru[b][r, pl.ds(0, L)] * ri[b][r, pl.ds(0, L)]
                for j in range(1, MW // L):
                    acc = acc + ru[b][r, pl.ds(j * L, L)] * ri[b][r, pl.ds(j * L, L)]
                for j in range(TW // L):
                    acc = acc + tu[b][r, pl.ds(j * L, L)] * ti[b][r, pl.ds(j * L, L)]
                accb[pl.ds(l * 17, L)] = acc
            tot = plsc.load_gather(accb, [lane17])
            for c in range(1, L):
                tot = tot + plsc.load_gather(accb, [lane17 + c])
            xv[pl.ds(off + g * L, L)] = tot
            return carry
        lax.fori_loop(0, CH // L, group, 0)

    def pair(t, carry):
        off0 = t * (2 * CH)
        off1 = off0 + CH
        h0 = issue(off0, 0)
        h1 = issue(off1, 1)
        for h in h0:
            h.wait()
        process(off0, 0)
        for h in h1:
            h.wait()
        process(off1, 1)
        return carry

    lax.fori_loop(0, NP, pair, 0)

    pltpu.sync_copy(xv, xui_hbm.at[pl.ds(base, BPW)])


def kernel(Gu, Gi, user, item):
    mesh = plsc.VectorSubcoreMesh(core_axis_name="c", subcore_axis_name="s")
    k = pl.kernel(
        _body,
        out_type=(
            jax.ShapeDtypeStruct((B,), jnp.float32),
            jax.ShapeDtypeStruct((B, DP), jnp.float32),
            jax.ShapeDtypeStruct((B, DP), jnp.float32),
        ),
        mesh=mesh,
        compiler_params=pltpu.CompilerParams(
            needs_layout_passes=False, use_tc_tiling_on_sc=True),
        scratch_types=(
            pltpu.VMEM((BPW,), jnp.int32),
            pltpu.VMEM((BPW,), jnp.int32),
            pltpu.VMEM((CH, MW), jnp.float32),
            pltpu.VMEM((CH, MW), jnp.float32),
            pltpu.VMEM((CH, MW), jnp.float32),
            pltpu.VMEM((CH, MW), jnp.float32),
            pltpu.VMEM((CH, TW), jnp.float32),
            pltpu.VMEM((CH, TW), jnp.float32),
            pltpu.VMEM((CH, TW), jnp.float32),
            pltpu.VMEM((CH, TW), jnp.float32),
            pltpu.VMEM((BPW,), jnp.float32),
            pltpu.VMEM((L * 17,), jnp.float32),
            pltpu.VMEM((CH, 128), jnp.float32),
            pltpu.VMEM((CH, 128), jnp.float32),
            pltpu.SemaphoreType.DMA,
            pltpu.SemaphoreType.DMA,
        ),
    )
    xui, gu, gi = k(Gu, Gi, user, item)
    return xui, gu[:, :D], gi[:, :D]


# speedup vs baseline: 1.0154x; 1.0154x over previous
import jax
import jax.numpy as jnp
from jax import lax
from jax.experimental import pallas as pl
from jax.experimental.pallas import tpu as pltpu
from jax.experimental.pallas import tpu_sc as plsc

NC = 2
NS = 16
L = 16
NW = NC * NS

D = 192
MW = 128
TW = D - MW
B = 16384
BPW = B // NW
CH = 64
NP = BPW // (2 * CH)


def _body(gu_hbm, gi_hbm, user_hbm, item_hbm,
          xui_hbm, gu_out, gi_out,
          idx_u, idx_i, ru0, ru1, ri0, ri1, tu0, tu1, ti0, ti1,
          xv, accb, s0, s1):
    cid = lax.axis_index("c")
    sid = lax.axis_index("s")
    wid = sid * NC + cid
    base = wid * BPW

    pltpu.sync_copy(user_hbm.at[pl.ds(base, BPW)], idx_u)
    pltpu.sync_copy(item_hbm.at[pl.ds(base, BPW)], idx_i)

    ru = (ru0, ru1)
    ri = (ri0, ri1)
    tu = (tu0, tu1)
    ti = (ti0, ti1)
    sems = (s0, s1)

    lane = lax.iota(jnp.int32, L)
    lane17 = lane * 17

    def issue(off, b):
        h = [
            pltpu.async_copy(
                gu_hbm.at[idx_u.at[pl.ds(off, CH)], pl.ds(0, MW)],
                ru[b], sems[b]),
            pltpu.async_copy(
                gi_hbm.at[idx_i.at[pl.ds(off, CH)], pl.ds(0, MW)],
                ri[b], sems[b]),
        ]
        for gg in range(CH // L):
            rvu = idx_u[pl.ds(off + gg * L, L)]
            rvi = idx_i[pl.ds(off + gg * L, L)]
            for l in range(L):
                kk = gg * L + l
                h.append(pltpu.async_copy(
                    gu_hbm.at[pl.ds(rvu[l], 1), pl.ds(MW, TW)],
                    tu[b].at[pl.ds(kk, 1)], sems[b]))
                h.append(pltpu.async_copy(
                    gi_hbm.at[pl.ds(rvi[l], 1), pl.ds(MW, TW)],
                    ti[b].at[pl.ds(kk, 1)], sems[b]))
        return h

    def process(off, b):
        pltpu.sync_copy(ru[b], gu_out.at[pl.ds(base + off, CH), pl.ds(0, MW)])
        pltpu.sync_copy(tu[b], gu_out.at[pl.ds(base + off, CH), pl.ds(MW, TW)])
        pltpu.sync_copy(ri[b], gi_out.at[pl.ds(base + off, CH), pl.ds(0, MW)])
        pltpu.sync_copy(ti[b], gi_out.at[pl.ds(base + off, CH), pl.ds(MW, TW)])

        def group(g, carry):
            for l in range(L):
                r = g * L + l
                acc = ru[b][r, pl.ds(0, L)] * ri[b][r, pl.ds(0, L)]
                for j in range(1, MW // L):
                    acc = acc + ru[b][r, pl.ds(j * L, L)] * ri[b][r, pl.ds(j * L, L)]
                for j in range(TW // L):
                    acc = acc + tu[b][r, pl.ds(j * L, L)] * ti[b][r, pl.ds(j * L, L)]
                accb[pl.ds(l * 17, L)] = acc
            tot = plsc.load_gather(accb, [lane17])
            for c in range(1, L):
                tot = tot + plsc.load_gather(accb, [lane17 + c])
            xv[pl.ds(off + g * L, L)] = tot
            return carry
        lax.fori_loop(0, CH // L, group, 0)

    def pair(t, carry):
        off0 = t * (2 * CH)
        off1 = off0 + CH
        h0 = issue(off0, 0)
        h1 = issue(off1, 1)
        for h in h0:
            h.wait()
        process(off0, 0)
        for h in h1:
            h.wait()
        process(off1, 1)
        return carry

    lax.fori_loop(0, NP, pair, 0)

    pltpu.sync_copy(xv, xui_hbm.at[pl.ds(base, BPW)])


def kernel(Gu, Gi, user, item):
    mesh = plsc.VectorSubcoreMesh(core_axis_name="c", subcore_axis_name="s")
    k = pl.kernel(
        _body,
        out_type=(
            jax.ShapeDtypeStruct((B,), jnp.float32),
            jax.ShapeDtypeStruct((B, D), jnp.float32),
            jax.ShapeDtypeStruct((B, D), jnp.float32),
        ),
        mesh=mesh,
        compiler_params=pltpu.CompilerParams(
            needs_layout_passes=False, use_tc_tiling_on_sc=True),
        scratch_types=(
            pltpu.VMEM((BPW,), jnp.int32),
            pltpu.VMEM((BPW,), jnp.int32),
            pltpu.VMEM((CH, MW), jnp.float32),
            pltpu.VMEM((CH, MW), jnp.float32),
            pltpu.VMEM((CH, MW), jnp.float32),
            pltpu.VMEM((CH, MW), jnp.float32),
            pltpu.VMEM((CH, TW), jnp.float32),
            pltpu.VMEM((CH, TW), jnp.float32),
            pltpu.VMEM((CH, TW), jnp.float32),
            pltpu.VMEM((CH, TW), jnp.float32),
            pltpu.VMEM((BPW,), jnp.float32),
            pltpu.VMEM((L * 17,), jnp.float32),
            pltpu.SemaphoreType.DMA,
            pltpu.SemaphoreType.DMA,
        ),
    )
    return k(Gu, Gi, user, item)
